# baseline (speedup 1.0000x reference)
import functools

import jax
import jax.numpy as jnp
from jax import lax
from jax.experimental import pallas as pl
from jax.experimental.pallas import tpu as pltpu
from jax.experimental.pallas import tpu_sc as plsc

B = 4096
M = 1000
MP = 1024
KNN = 4
NEG = -3.0e38

_BR0 = 512


def _norm_body(x_ref, o_ref):
    x = x_ref[...]
    ssq = jnp.sum(x * x, axis=1, keepdims=True)
    nrm = jnp.maximum(jnp.sqrt(ssq), 1e-12)
    o_ref[...] = (x / nrm).astype(jnp.bfloat16)


def _normalize(logits_t):
    return pl.pallas_call(
        _norm_body,
        grid=(B // _BR0,),
        in_specs=[pl.BlockSpec((_BR0, M), lambda i: (i, 0))],
        out_specs=pl.BlockSpec((_BR0, M), lambda i: (i, 0)),
        out_shape=jax.ShapeDtypeStruct((B, M), jnp.bfloat16),
    )(logits_t)


_BR1 = 512


def _topk_body(row0, lhs_ref, tn_ref, o_ref):
    g = lax.dot_general(lhs_ref[...].astype(jnp.bfloat16), tn_ref[...],
                        (((1,), (1,)), ((), ())),
                        preferred_element_type=jnp.float32)
    r0 = pl.program_id(0) * _BR1 + row0
    col = lax.broadcasted_iota(jnp.int32, (_BR1, B), 1)
    row = lax.broadcasted_iota(jnp.int32, (_BR1, B), 0) + r0
    bits = lax.bitcast_convert_type(g, jnp.int32)
    key = bits ^ ((bits >> 31) & jnp.int32(0x7FFFFFFF))
    packed = (key & jnp.int32(~0xFFF)) | (jnp.int32(B - 1) - col)
    imin = jnp.int32(-0x80000000)
    packed = jnp.where(col == row, imin, packed)
    lane = lax.broadcasted_iota(jnp.int32, (_BR1, 128), 1)
    out = jnp.zeros((_BR1, 128), jnp.int32)
    for k in range(KNN):
        m = jnp.max(packed, axis=1, keepdims=True)
        am = jnp.int32(B - 1) - (m & jnp.int32(0xFFF))
        out = jnp.where(lane == k, am, out)
        packed = jnp.where(packed == m, imin, packed)
    o_ref[...] = out


def _top4(logits_t, t_norm, row0, nrows):
    blk0 = row0 // _BR1
    return pl.pallas_call(
        functools.partial(_topk_body, row0),
        grid=(nrows // _BR1,),
        in_specs=[
            pl.BlockSpec((_BR1, M), lambda i: (i + blk0, 0)),
            pl.BlockSpec((B, M), lambda i: (0, 0)),
        ],
        out_specs=pl.BlockSpec((_BR1, 128), lambda i: (i, 0)),
        out_shape=jax.ShapeDtypeStruct((nrows, 128), jnp.int32),
    )(logits_t, t_norm)


_CH = 8
_NB = 4
_LA = 2


def _make_gather(nrows):
    info = plsc.get_sparse_core_info()
    nw = info.num_cores * info.num_subcores
    bpw = nrows // nw
    nch = bpw // _CH
    mesh = plsc.VectorSubcoreMesh(core_axis_name="c", subcore_axis_name="s")

    @functools.partial(
        pl.kernel,
        mesh=mesh,
        out_type=(jax.ShapeDtypeStruct((nrows, MP // 2), jnp.int32),
                  jax.ShapeDtypeStruct((nrows, MP // 2), jnp.int32)),
        scratch_types=[
            pltpu.VMEM((bpw,), jnp.int32),
            pltpu.VMEM((_NB, _CH, MP // 2), jnp.int32),
            pltpu.VMEM((_NB, _CH, MP // 2), jnp.int32),
            pltpu.SemaphoreType.DMA,
            pltpu.SemaphoreType.DMA,
            pltpu.SemaphoreType.DMA,
            pltpu.SemaphoreType.DMA,
        ],
    )
    def gather(ts_hbm, tt_hbm, idx_hbm, os_hbm, ot_hbm,
               idx_v, bufs, buft, gsem_s, gsem_t, osem_s, osem_t):
        wid = lax.axis_index("s") * info.num_cores + lax.axis_index("c")
        base = wid * bpw
        pltpu.sync_copy(idx_hbm.at[pl.ds(base, bpw)], idx_v)

        def fire_gathers(c, slot):
            ic = idx_v.at[pl.ds(c * _CH, _CH)]
            pltpu.async_copy(ts_hbm.at[ic], bufs.at[slot], gsem_s)
            pltpu.async_copy(tt_hbm.at[ic], buft.at[slot], gsem_t)

        def drain_g(slot):
            pltpu.make_async_copy(ts_hbm.at[pl.ds(0, _CH)], bufs.at[slot],
                                  gsem_s).wait()
            pltpu.make_async_copy(tt_hbm.at[pl.ds(0, _CH)], buft.at[slot],
                                  gsem_t).wait()

        def drain_o(slot):
            pltpu.make_async_copy(bufs.at[slot], os_hbm.at[pl.ds(0, _CH)],
                                  osem_s).wait()
            pltpu.make_async_copy(buft.at[slot], ot_hbm.at[pl.ds(0, _CH)],
                                  osem_t).wait()

        for c in range(_LA):
            fire_gathers(c, c % _NB)

        def body(i, _):
            for j in range(_NB):
                c = i * _NB + j
                slot = j
                off = base + c * _CH
                drain_g(slot)
                pltpu.async_copy(bufs.at[slot], os_hbm.at[pl.ds(off, _CH)],
                                 osem_s)
                pltpu.async_copy(buft.at[slot], ot_hbm.at[pl.ds(off, _CH)],
                                 osem_t)
                cn = c + _LA

                @pl.when(cn >= _NB)
                def _():
                    drain_o((slot + _LA) % _NB)

                @pl.when(cn < nch)
                def _():
                    fire_gathers(cn, (slot + _LA) % _NB)

            return 0

        lax.fori_loop(0, nch // _NB, body, 0)
        for _ in range(_LA):
            drain_o(0)

    return gather


_BR3 = 256
_SCALE = 0.5 / (B * KNN)


def _js_body(ls_ref, lt_ref, gs_ref, gt_ref, o_ref, acc_ref):
    @pl.when(pl.program_id(0) == 0)
    def _():
        acc_ref[0, 0] = 0.0

    ls = ls_ref[...]
    lt = lt_ref[...]
    total = jnp.float32(0.0)
    for k in range(KNN):
        dq = ls - gs_ref[k][:, :M].astype(jnp.float32)
        dp = lt - gt_ref[k][:, :M].astype(jnp.float32)

        def softmax_logsoftmax(d):
            m = jnp.max(d, axis=1, keepdims=True)
            e = jnp.exp(d - m)
            s = jnp.sum(e, axis=1, keepdims=True)
            return e / s, (d - m) - jnp.log(s)

        q, logq = softmax_logsoftmax(dq)
        p, logp = softmax_logsoftmax(dp)
        logmean = jnp.log((q + p) * 0.5)
        contrib = q * (logq - logmean) + p * (logp - logmean)
        total = total + jnp.sum(contrib)
    acc_ref[0, 0] += total

    @pl.when(pl.program_id(0) == pl.num_programs(0) - 1)
    def _():
        o_ref[0, 0] = acc_ref[0, 0] * _SCALE


def _js_loss(ls, lt, gs, gt, row0, nrows):
    blk0 = row0 // _BR3
    out = pl.pallas_call(
        _js_body,
        grid=(nrows // _BR3,),
        in_specs=[
            pl.BlockSpec((_BR3, M), lambda i: (i + blk0, 0)),
            pl.BlockSpec((_BR3, M), lambda i: (i + blk0, 0)),
            pl.BlockSpec((KNN, _BR3, MP), lambda i: (0, i, 0)),
            pl.BlockSpec((KNN, _BR3, MP), lambda i: (0, i, 0)),
        ],
        out_specs=pl.BlockSpec(memory_space=pltpu.SMEM),
        out_shape=jax.ShapeDtypeStruct((1, 1), jnp.float32),
        scratch_shapes=[pltpu.SMEM((1, 1), jnp.float32)],
    )(ls, lt, gs, gt)
    return out[0, 0]


_NSPLIT = 2
_HB = B // _NSPLIT


def kernel(logits_s, logits_t):
    t_norm = _normalize(logits_t)
    pad = ((0, 0), (0, MP - M))

    def to_i32(x):
        xb = jnp.pad(x, pad).astype(jnp.bfloat16).reshape(B, MP // 2, 2)
        return lax.bitcast_convert_type(xb, jnp.int32)

    def from_i32(x, nrows):
        xb = lax.bitcast_convert_type(x, jnp.bfloat16)
        return xb.reshape(KNN, nrows // KNN, MP)

    ls_b = to_i32(logits_s)
    lt_b = to_i32(logits_t)
    gather = _make_gather(_HB * KNN)

    partials = []
    for h in range(_NSPLIT):
        idx128 = _top4(logits_t, t_norm, h * _HB, _HB)
        flat_idx = idx128[:, :KNN].T.reshape(-1)
        gs_flat, gt_flat = gather(ls_b, lt_b, flat_idx)
        gs = from_i32(gs_flat, _HB * KNN)
        gt = from_i32(gt_flat, _HB * KNN)
        partials.append(_js_loss(logits_s, logits_t, gs, gt, h * _HB, _HB))
    return sum(partials)


# speedup vs baseline: 3.0505x; 3.0505x over previous
import functools

import jax
import jax.numpy as jnp
from jax import lax
from jax.experimental import pallas as pl
from jax.experimental.pallas import tpu as pltpu
from jax.experimental.pallas import tpu_sc as plsc

B = 4096
M = 1000
MP = 1024
KNN = 4
NEG = -3.0e38

_BR0 = 512


def _norm_body(x_ref, o_ref):
    x = x_ref[...]
    ssq = jnp.sum(x * x, axis=1, keepdims=True)
    nrm = jnp.maximum(jnp.sqrt(ssq), 1e-12)
    o_ref[...] = (x / nrm).astype(jnp.bfloat16)


def _normalize(logits_t):
    return pl.pallas_call(
        _norm_body,
        grid=(B // _BR0,),
        in_specs=[pl.BlockSpec((_BR0, M), lambda i: (i, 0))],
        out_specs=pl.BlockSpec((_BR0, M), lambda i: (i, 0)),
        out_shape=jax.ShapeDtypeStruct((B, M), jnp.bfloat16),
    )(logits_t)


_BR1 = 512


def _topk_body(row0, lhs_ref, tn_ref, o_ref):
    g = lax.dot_general(lhs_ref[...].astype(jnp.bfloat16), tn_ref[...],
                        (((1,), (1,)), ((), ())),
                        preferred_element_type=jnp.float32)
    r0 = pl.program_id(0) * _BR1 + row0
    col = lax.broadcasted_iota(jnp.int32, (_BR1, B), 1)
    row = lax.broadcasted_iota(jnp.int32, (_BR1, B), 0) + r0
    bits = lax.bitcast_convert_type(g, jnp.int32)
    key = bits ^ ((bits >> 31) & jnp.int32(0x7FFFFFFF))
    packed = (key & jnp.int32(~0xFFF)) | (jnp.int32(B - 1) - col)
    imin = jnp.int32(-0x80000000)
    packed = jnp.where(col == row, imin, packed)
    lane = lax.broadcasted_iota(jnp.int32, (_BR1, 128), 1)
    out = jnp.zeros((_BR1, 128), jnp.int32)
    for k in range(KNN):
        m = jnp.max(packed, axis=1, keepdims=True)
        am = jnp.int32(B - 1) - (m & jnp.int32(0xFFF))
        out = jnp.where(lane == k, am, out)
        packed = jnp.where(packed == m, imin, packed)
    o_ref[...] = out


def _top4(logits_t, t_norm, row0, nrows):
    blk0 = row0 // _BR1
    return pl.pallas_call(
        functools.partial(_topk_body, row0),
        grid=(nrows // _BR1,),
        in_specs=[
            pl.BlockSpec((_BR1, M), lambda i: (i + blk0, 0)),
            pl.BlockSpec((B, M), lambda i: (0, 0)),
        ],
        out_specs=pl.BlockSpec((_BR1, 128), lambda i: (i, 0)),
        out_shape=jax.ShapeDtypeStruct((nrows, 128), jnp.int32),
    )(logits_t, t_norm)


_CH = 8
_NB = 4
_LA = 2


def _make_gather(nrows):
    info = plsc.get_sparse_core_info()
    nw = info.num_cores * info.num_subcores
    bpw = nrows // nw
    nch = bpw // _CH
    mesh = plsc.VectorSubcoreMesh(core_axis_name="c", subcore_axis_name="s")

    @functools.partial(
        pl.kernel,
        mesh=mesh,
        out_type=(jax.ShapeDtypeStruct((nrows, MP), jnp.float32),
                  jax.ShapeDtypeStruct((nrows, MP), jnp.float32)),
        scratch_types=[
            pltpu.VMEM((bpw,), jnp.int32),
            pltpu.VMEM((_NB, _CH, MP), jnp.float32),
            pltpu.VMEM((_NB, _CH, MP), jnp.float32),
            pltpu.SemaphoreType.DMA,
            pltpu.SemaphoreType.DMA,
            pltpu.SemaphoreType.DMA,
            pltpu.SemaphoreType.DMA,
        ],
    )
    def gather(ts_hbm, tt_hbm, idx_hbm, os_hbm, ot_hbm,
               idx_v, bufs, buft, gsem_s, gsem_t, osem_s, osem_t):
        wid = lax.axis_index("s") * info.num_cores + lax.axis_index("c")
        base = wid * bpw
        pltpu.sync_copy(idx_hbm.at[pl.ds(base, bpw)], idx_v)

        def fire_gathers(c, slot):
            ic = idx_v.at[pl.ds(c * _CH, _CH)]
            pltpu.async_copy(ts_hbm.at[ic], bufs.at[slot], gsem_s)
            pltpu.async_copy(tt_hbm.at[ic], buft.at[slot], gsem_t)

        def drain_g(slot):
            pltpu.make_async_copy(ts_hbm.at[pl.ds(0, _CH)], bufs.at[slot],
                                  gsem_s).wait()
            pltpu.make_async_copy(tt_hbm.at[pl.ds(0, _CH)], buft.at[slot],
                                  gsem_t).wait()

        def drain_o(slot):
            pltpu.make_async_copy(bufs.at[slot], os_hbm.at[pl.ds(0, _CH)],
                                  osem_s).wait()
            pltpu.make_async_copy(buft.at[slot], ot_hbm.at[pl.ds(0, _CH)],
                                  osem_t).wait()

        for c in range(_LA):
            fire_gathers(c, c % _NB)

        def body(i, _):
            for j in range(_NB):
                c = i * _NB + j
                slot = j
                off = base + c * _CH
                drain_g(slot)
                pltpu.async_copy(bufs.at[slot], os_hbm.at[pl.ds(off, _CH)],
                                 osem_s)
                pltpu.async_copy(buft.at[slot], ot_hbm.at[pl.ds(off, _CH)],
                                 osem_t)
                cn = c + _LA

                @pl.when(cn >= _NB)
                def _():
                    drain_o((slot + _LA) % _NB)

                @pl.when(cn < nch)
                def _():
                    fire_gathers(cn, (slot + _LA) % _NB)

            return 0

        lax.fori_loop(0, nch // _NB, body, 0)
        for _ in range(_LA):
            drain_o(0)

    return gather


_BR3 = 256
_SCALE = 0.5 / (B * KNN)


def _js_body(ls_ref, lt_ref, gs_ref, gt_ref, o_ref, acc_ref):
    @pl.when(pl.program_id(0) == 0)
    def _():
        acc_ref[0, 0] = 0.0

    ls = ls_ref[...]
    lt = lt_ref[...]
    total = jnp.float32(0.0)
    for k in range(KNN):
        dq = ls - gs_ref[k][:, :M].astype(jnp.float32)
        dp = lt - gt_ref[k][:, :M].astype(jnp.float32)

        def softmax_logsoftmax(d):
            m = jnp.max(d, axis=1, keepdims=True)
            e = jnp.exp(d - m)
            s = jnp.sum(e, axis=1, keepdims=True)
            return e / s, (d - m) - jnp.log(s)

        q, logq = softmax_logsoftmax(dq)
        p, logp = softmax_logsoftmax(dp)
        logmean = jnp.log((q + p) * 0.5)
        contrib = q * (logq - logmean) + p * (logp - logmean)
        total = total + jnp.sum(contrib)
    acc_ref[0, 0] += total

    @pl.when(pl.program_id(0) == pl.num_programs(0) - 1)
    def _():
        o_ref[0, 0] = acc_ref[0, 0] * _SCALE


def _js_loss(ls, lt, gs, gt, row0, nrows):
    blk0 = row0 // _BR3
    out = pl.pallas_call(
        _js_body,
        grid=(nrows // _BR3,),
        in_specs=[
            pl.BlockSpec((_BR3, M), lambda i: (i + blk0, 0)),
            pl.BlockSpec((_BR3, M), lambda i: (i + blk0, 0)),
            pl.BlockSpec((KNN, _BR3, MP), lambda i: (0, i, 0)),
            pl.BlockSpec((KNN, _BR3, MP), lambda i: (0, i, 0)),
        ],
        out_specs=pl.BlockSpec(memory_space=pltpu.SMEM),
        out_shape=jax.ShapeDtypeStruct((1, 1), jnp.float32),
        scratch_shapes=[pltpu.SMEM((1, 1), jnp.float32)],
    )(ls, lt, gs, gt)
    return out[0, 0]


_NSPLIT = 2
_HB = B // _NSPLIT


def kernel(logits_s, logits_t):
    t_norm = _normalize(logits_t)
    ls_p = jnp.pad(logits_s, ((0, 0), (0, MP - M)))
    lt_p = jnp.pad(logits_t, ((0, 0), (0, MP - M)))
    gather = _make_gather(_HB * KNN)

    partials = []
    for h in range(_NSPLIT):
        idx128 = _top4(logits_t, t_norm, h * _HB, _HB)
        flat_idx = idx128[:, :KNN].T.reshape(-1)
        gs_flat, gt_flat = gather(ls_p, lt_p, flat_idx)
        gs = gs_flat.reshape(KNN, _HB, MP)
        gt = gt_flat.reshape(KNN, _HB, MP)
        partials.append(_js_loss(logits_s, logits_t, gs, gt, h * _HB, _HB))
    return sum(partials)


# speedup vs baseline: 3.0541x; 1.0012x over previous
import functools

import jax
import jax.numpy as jnp
from jax import lax
from jax.experimental import pallas as pl
from jax.experimental.pallas import tpu as pltpu
from jax.experimental.pallas import tpu_sc as plsc

B = 4096
M = 1000
MP = 1024
KNN = 4
NEG = -3.0e38

_BR0 = 512


def _norm_body(x_ref, o_ref):
    x = x_ref[...]
    ssq = jnp.sum(x * x, axis=1, keepdims=True)
    nrm = jnp.maximum(jnp.sqrt(ssq), 1e-12)
    o_ref[...] = (x / nrm).astype(jnp.bfloat16)


def _normalize(logits_t):
    return pl.pallas_call(
        _norm_body,
        grid=(B // _BR0,),
        in_specs=[pl.BlockSpec((_BR0, M), lambda i: (i, 0))],
        out_specs=pl.BlockSpec((_BR0, M), lambda i: (i, 0)),
        out_shape=jax.ShapeDtypeStruct((B, M), jnp.bfloat16),
    )(logits_t)


_BR1 = 512


def _topk_body(row0, lhs_ref, tn_ref, o_ref):
    g = lax.dot_general(lhs_ref[...].astype(jnp.bfloat16), tn_ref[...],
                        (((1,), (1,)), ((), ())),
                        preferred_element_type=jnp.float32)
    r0 = pl.program_id(0) * _BR1 + row0
    col = lax.broadcasted_iota(jnp.int32, (_BR1, B), 1)
    row = lax.broadcasted_iota(jnp.int32, (_BR1, B), 0) + r0
    bits = lax.bitcast_convert_type(g, jnp.int32)
    key = bits ^ ((bits >> 31) & jnp.int32(0x7FFFFFFF))
    packed = (key & jnp.int32(~0xFFF)) | (jnp.int32(B - 1) - col)
    imin = jnp.int32(-0x80000000)
    packed = jnp.where(col == row, imin, packed)
    lane = lax.broadcasted_iota(jnp.int32, (_BR1, 128), 1)
    out = jnp.zeros((_BR1, 128), jnp.int32)
    for k in range(KNN):
        m = jnp.max(packed, axis=1, keepdims=True)
        am = jnp.int32(B - 1) - (m & jnp.int32(0xFFF))
        out = jnp.where(lane == k, am, out)
        packed = jnp.where(packed == m, imin, packed)
    o_ref[...] = out


def _top4(logits_t, t_norm, row0, nrows):
    blk0 = row0 // _BR1
    return pl.pallas_call(
        functools.partial(_topk_body, row0),
        grid=(nrows // _BR1,),
        in_specs=[
            pl.BlockSpec((_BR1, M), lambda i: (i + blk0, 0)),
            pl.BlockSpec((B, M), lambda i: (0, 0)),
        ],
        out_specs=pl.BlockSpec((_BR1, 128), lambda i: (i, 0)),
        out_shape=jax.ShapeDtypeStruct((nrows, 128), jnp.int32),
    )(logits_t, t_norm)


_CH = 8
_NB = 4
_LA = 2


def _make_gather(nrows):
    info = plsc.get_sparse_core_info()
    nw = info.num_cores * info.num_subcores
    bpw = nrows // nw
    nch = bpw // _CH
    mesh = plsc.VectorSubcoreMesh(core_axis_name="c", subcore_axis_name="s")

    @functools.partial(
        pl.kernel,
        mesh=mesh,
        out_type=(jax.ShapeDtypeStruct((nrows, MP), jnp.float32),
                  jax.ShapeDtypeStruct((nrows, MP), jnp.float32)),
        scratch_types=[
            pltpu.VMEM((bpw,), jnp.int32),
            pltpu.VMEM((_NB, _CH, MP), jnp.float32),
            pltpu.VMEM((_NB, _CH, MP), jnp.float32),
            pltpu.SemaphoreType.DMA,
            pltpu.SemaphoreType.DMA,
            pltpu.SemaphoreType.DMA,
            pltpu.SemaphoreType.DMA,
        ],
        compiler_params=pltpu.CompilerParams(use_tc_tiling_on_sc=True),
    )
    def gather(ts_hbm, tt_hbm, idx_hbm, os_hbm, ot_hbm,
               idx_v, bufs, buft, gsem_s, gsem_t, osem_s, osem_t):
        wid = lax.axis_index("s") * info.num_cores + lax.axis_index("c")
        base = wid * bpw
        pltpu.sync_copy(idx_hbm.at[pl.ds(base, bpw)], idx_v)

        def fire_gathers(c, slot):
            ic = idx_v.at[pl.ds(c * _CH, _CH)]
            pltpu.async_copy(ts_hbm.at[ic], bufs.at[slot], gsem_s)
            pltpu.async_copy(tt_hbm.at[ic], buft.at[slot], gsem_t)

        def drain_g(slot):
            pltpu.make_async_copy(ts_hbm.at[pl.ds(0, _CH)], bufs.at[slot],
                                  gsem_s).wait()
            pltpu.make_async_copy(tt_hbm.at[pl.ds(0, _CH)], buft.at[slot],
                                  gsem_t).wait()

        def drain_o(slot):
            pltpu.make_async_copy(bufs.at[slot], os_hbm.at[pl.ds(0, _CH)],
                                  osem_s).wait()
            pltpu.make_async_copy(buft.at[slot], ot_hbm.at[pl.ds(0, _CH)],
                                  osem_t).wait()

        for c in range(_LA):
            fire_gathers(c, c % _NB)

        def body(i, _):
            for j in range(_NB):
                c = i * _NB + j
                slot = j
                off = base + c * _CH
                drain_g(slot)
                pltpu.async_copy(bufs.at[slot], os_hbm.at[pl.ds(off, _CH)],
                                 osem_s)
                pltpu.async_copy(buft.at[slot], ot_hbm.at[pl.ds(off, _CH)],
                                 osem_t)
                cn = c + _LA

                @pl.when(cn >= _NB)
                def _():
                    drain_o((slot + _LA) % _NB)

                @pl.when(cn < nch)
                def _():
                    fire_gathers(cn, (slot + _LA) % _NB)

            return 0

        lax.fori_loop(0, nch // _NB, body, 0)
        for _ in range(_LA):
            drain_o(0)

    return gather


_BR3 = 256
_SCALE = 0.5 / (B * KNN)


def _js_body(ls_ref, lt_ref, gs_ref, gt_ref, o_ref, acc_ref):
    @pl.when(pl.program_id(0) == 0)
    def _():
        acc_ref[0, 0] = 0.0

    ls = ls_ref[...]
    lt = lt_ref[...]
    total = jnp.float32(0.0)
    for k in range(KNN):
        dq = ls - gs_ref[k][:, :M].astype(jnp.float32)
        dp = lt - gt_ref[k][:, :M].astype(jnp.float32)

        def softmax_logsoftmax(d):
            m = jnp.max(d, axis=1, keepdims=True)
            e = jnp.exp(d - m)
            s = jnp.sum(e, axis=1, keepdims=True)
            return e / s, (d - m) - jnp.log(s)

        q, logq = softmax_logsoftmax(dq)
        p, logp = softmax_logsoftmax(dp)
        logmean = jnp.log((q + p) * 0.5)
        contrib = q * (logq - logmean) + p * (logp - logmean)
        total = total + jnp.sum(contrib)
    acc_ref[0, 0] += total

    @pl.when(pl.program_id(0) == pl.num_programs(0) - 1)
    def _():
        o_ref[0, 0] = acc_ref[0, 0] * _SCALE


def _js_loss(ls, lt, gs, gt, row0, nrows):
    blk0 = row0 // _BR3
    out = pl.pallas_call(
        _js_body,
        grid=(nrows // _BR3,),
        in_specs=[
            pl.BlockSpec((_BR3, M), lambda i: (i + blk0, 0)),
            pl.BlockSpec((_BR3, M), lambda i: (i + blk0, 0)),
            pl.BlockSpec((KNN, _BR3, MP), lambda i: (0, i, 0)),
            pl.BlockSpec((KNN, _BR3, MP), lambda i: (0, i, 0)),
        ],
        out_specs=pl.BlockSpec(memory_space=pltpu.SMEM),
        out_shape=jax.ShapeDtypeStruct((1, 1), jnp.float32),
        scratch_shapes=[pltpu.SMEM((1, 1), jnp.float32)],
    )(ls, lt, gs, gt)
    return out[0, 0]


_NSPLIT = 2
_HB = B // _NSPLIT


def kernel(logits_s, logits_t):
    t_norm = _normalize(logits_t)
    ls_p = jnp.pad(logits_s, ((0, 0), (0, MP - M)))
    lt_p = jnp.pad(logits_t, ((0, 0), (0, MP - M)))
    gather = _make_gather(_HB * KNN)

    partials = []
    for h in range(_NSPLIT):
        idx128 = _top4(logits_t, t_norm, h * _HB, _HB)
        flat_idx = idx128[:, :KNN].T.reshape(-1)
        gs_flat, gt_flat = gather(ls_p, lt_p, flat_idx)
        gs = gs_flat.reshape(KNN, _HB, MP)
        gt = gt_flat.reshape(KNN, _HB, MP)
        partials.append(_js_loss(logits_s, logits_t, gs, gt, h * _HB, _HB))
    return sum(partials)


# speedup vs baseline: 3.1529x; 1.0323x over previous
import functools

import jax
import jax.numpy as jnp
from jax import lax
from jax.experimental import pallas as pl
from jax.experimental.pallas import tpu as pltpu
from jax.experimental.pallas import tpu_sc as plsc

B = 4096
M = 1000
MP = 1024
KNN = 4
NEG = -3.0e38

_BR0 = 512


def _norm_body(ls_ref, lt_ref, tn_ref, lsp_ref, ltp_ref):
    zpad = jnp.zeros((_BR0, MP - M), jnp.float32)
    lt = lt_ref[...]
    ssq = jnp.sum(lt * lt, axis=1, keepdims=True)
    nrm = jnp.maximum(jnp.sqrt(ssq), 1e-12)
    tn_ref[...] = (lt / nrm).astype(jnp.bfloat16)
    lsp_ref[...] = jnp.concatenate([ls_ref[...], zpad], axis=1)
    ltp_ref[...] = jnp.concatenate([lt, zpad], axis=1)


def _normalize(logits_s, logits_t):
    return pl.pallas_call(
        _norm_body,
        grid=(B // _BR0,),
        in_specs=[
            pl.BlockSpec((_BR0, M), lambda i: (i, 0)),
            pl.BlockSpec((_BR0, M), lambda i: (i, 0)),
        ],
        out_specs=[
            pl.BlockSpec((_BR0, M), lambda i: (i, 0)),
            pl.BlockSpec((_BR0, MP), lambda i: (i, 0)),
            pl.BlockSpec((_BR0, MP), lambda i: (i, 0)),
        ],
        out_shape=[
            jax.ShapeDtypeStruct((B, M), jnp.bfloat16),
            jax.ShapeDtypeStruct((B, MP), jnp.float32),
            jax.ShapeDtypeStruct((B, MP), jnp.float32),
        ],
    )(logits_s, logits_t)


_BR1 = 512


def _topk_body(row0, lhs_ref, tn_ref, o_ref):
    g = lax.dot_general(lhs_ref[...].astype(jnp.bfloat16), tn_ref[...],
                        (((1,), (1,)), ((), ())),
                        preferred_element_type=jnp.float32)
    r0 = pl.program_id(0) * _BR1 + row0
    col = lax.broadcasted_iota(jnp.int32, (_BR1, B), 1)
    row = lax.broadcasted_iota(jnp.int32, (_BR1, B), 0) + r0
    bits = lax.bitcast_convert_type(g, jnp.int32)
    key = bits ^ ((bits >> 31) & jnp.int32(0x7FFFFFFF))
    packed = (key & jnp.int32(~0xFFF)) | (jnp.int32(B - 1) - col)
    imin = jnp.int32(-0x80000000)
    packed = jnp.where(col == row, imin, packed)
    lane = lax.broadcasted_iota(jnp.int32, (_BR1, 128), 1)
    out = jnp.zeros((_BR1, 128), jnp.int32)
    for k in range(KNN):
        m = jnp.max(packed, axis=1, keepdims=True)
        am = jnp.int32(B - 1) - (m & jnp.int32(0xFFF))
        out = jnp.where(lane == k, am, out)
        packed = jnp.where(packed == m, imin, packed)
    o_ref[...] = out


def _top4(logits_t, t_norm, row0, nrows):
    blk0 = row0 // _BR1
    return pl.pallas_call(
        functools.partial(_topk_body, row0),
        grid=(nrows // _BR1,),
        in_specs=[
            pl.BlockSpec((_BR1, M), lambda i: (i + blk0, 0)),
            pl.BlockSpec((B, M), lambda i: (0, 0)),
        ],
        out_specs=pl.BlockSpec((_BR1, 128), lambda i: (i, 0)),
        out_shape=jax.ShapeDtypeStruct((nrows, 128), jnp.int32),
    )(logits_t, t_norm)


_CH = 8
_NB = 4
_LA = 2


def _make_gather(nrows):
    info = plsc.get_sparse_core_info()
    nw = info.num_cores * info.num_subcores
    bpw = nrows // nw
    nch = bpw // _CH
    mesh = plsc.VectorSubcoreMesh(core_axis_name="c", subcore_axis_name="s")

    @functools.partial(
        pl.kernel,
        mesh=mesh,
        out_type=(jax.ShapeDtypeStruct((nrows, MP), jnp.float32),
                  jax.ShapeDtypeStruct((nrows, MP), jnp.float32)),
        scratch_types=[
            pltpu.VMEM((bpw,), jnp.int32),
            pltpu.VMEM((_NB, _CH, MP), jnp.float32),
            pltpu.VMEM((_NB, _CH, MP), jnp.float32),
            pltpu.SemaphoreType.DMA,
            pltpu.SemaphoreType.DMA,
            pltpu.SemaphoreType.DMA,
            pltpu.SemaphoreType.DMA,
        ],
    )
    def gather(ts_hbm, tt_hbm, idx_hbm, os_hbm, ot_hbm,
               idx_v, bufs, buft, gsem_s, gsem_t, osem_s, osem_t):
        wid = lax.axis_index("s") * info.num_cores + lax.axis_index("c")
        base = wid * bpw
        pltpu.sync_copy(idx_hbm.at[pl.ds(base, bpw)], idx_v)

        def fire_gathers(c, slot):
            ic = idx_v.at[pl.ds(c * _CH, _CH)]
            pltpu.async_copy(ts_hbm.at[ic], bufs.at[slot], gsem_s)
            pltpu.async_copy(tt_hbm.at[ic], buft.at[slot], gsem_t)

        def drain_g(slot):
            pltpu.make_async_copy(ts_hbm.at[pl.ds(0, _CH)], bufs.at[slot],
                                  gsem_s).wait()
            pltpu.make_async_copy(tt_hbm.at[pl.ds(0, _CH)], buft.at[slot],
                                  gsem_t).wait()

        def drain_o(slot):
            pltpu.make_async_copy(bufs.at[slot], os_hbm.at[pl.ds(0, _CH)],
                                  osem_s).wait()
            pltpu.make_async_copy(buft.at[slot], ot_hbm.at[pl.ds(0, _CH)],
                                  osem_t).wait()

        for c in range(_LA):
            fire_gathers(c, c % _NB)

        def body(i, _):
            for j in range(_NB):
                c = i * _NB + j
                slot = j
                off = base + c * _CH
                drain_g(slot)
                pltpu.async_copy(bufs.at[slot], os_hbm.at[pl.ds(off, _CH)],
                                 osem_s)
                pltpu.async_copy(buft.at[slot], ot_hbm.at[pl.ds(off, _CH)],
                                 osem_t)
                cn = c + _LA

                @pl.when(cn >= _NB)
                def _():
                    drain_o((slot + _LA) % _NB)

                @pl.when(cn < nch)
                def _():
                    fire_gathers(cn, (slot + _LA) % _NB)

            return 0

        lax.fori_loop(0, nch // _NB, body, 0)
        for _ in range(_LA):
            drain_o(0)

    return gather


_BR3 = 256
_SCALE = 0.5 / (B * KNN)


def _js_body(ls_ref, lt_ref, gs_ref, gt_ref, o_ref, acc_ref):
    @pl.when(pl.program_id(0) == 0)
    def _():
        acc_ref[0, 0] = 0.0

    ls = ls_ref[...]
    lt = lt_ref[...]
    total = jnp.float32(0.0)
    for k in range(KNN):
        dq = ls - gs_ref[k][:, :M].astype(jnp.float32)
        dp = lt - gt_ref[k][:, :M].astype(jnp.float32)

        def softmax_logsoftmax(d):
            m = jnp.max(d, axis=1, keepdims=True)
            e = jnp.exp(d - m)
            s = jnp.sum(e, axis=1, keepdims=True)
            return e / s, (d - m) - jnp.log(s)

        q, logq = softmax_logsoftmax(dq)
        p, logp = softmax_logsoftmax(dp)
        logmean = jnp.log((q + p) * 0.5)
        contrib = q * (logq - logmean) + p * (logp - logmean)
        total = total + jnp.sum(contrib)
    acc_ref[0, 0] += total

    @pl.when(pl.program_id(0) == pl.num_programs(0) - 1)
    def _():
        o_ref[0, 0] = acc_ref[0, 0] * _SCALE


def _js_loss(ls, lt, gs, gt, row0, nrows):
    blk0 = row0 // _BR3
    out = pl.pallas_call(
        _js_body,
        grid=(nrows // _BR3,),
        in_specs=[
            pl.BlockSpec((_BR3, M), lambda i: (i + blk0, 0)),
            pl.BlockSpec((_BR3, M), lambda i: (i + blk0, 0)),
            pl.BlockSpec((KNN, _BR3, MP), lambda i: (0, i, 0)),
            pl.BlockSpec((KNN, _BR3, MP), lambda i: (0, i, 0)),
        ],
        out_specs=pl.BlockSpec(memory_space=pltpu.SMEM),
        out_shape=jax.ShapeDtypeStruct((1, 1), jnp.float32),
        scratch_shapes=[pltpu.SMEM((1, 1), jnp.float32)],
    )(ls, lt, gs, gt)
    return out[0, 0]


_NSPLIT = 2
_HB = B // _NSPLIT


def kernel(logits_s, logits_t):
    t_norm, ls_p, lt_p = _normalize(logits_s, logits_t)
    gather = _make_gather(_HB * KNN)

    partials = []
    for h in range(_NSPLIT):
        idx128 = _top4(logits_t, t_norm, h * _HB, _HB)
        flat_idx = idx128[:, :KNN].T.reshape(-1)
        gs_flat, gt_flat = gather(ls_p, lt_p, flat_idx)
        gs = gs_flat.reshape(KNN, _HB, MP)
        gt = gt_flat.reshape(KNN, _HB, MP)
        partials.append(_js_loss(logits_s, logits_t, gs, gt, h * _HB, _HB))
    return sum(partials)


# speedup vs baseline: 3.6728x; 1.1649x over previous
import functools

import jax
import jax.numpy as jnp
from jax import lax
from jax.experimental import pallas as pl
from jax.experimental.pallas import tpu as pltpu
from jax.experimental.pallas import tpu_sc as plsc

B = 4096
M = 1000
MP = 1024
KNN = 4
NEG = -3.0e38

_BR0 = 512


def _rnd_bf16_bits(x):
    b = lax.bitcast_convert_type(x, jnp.int32)
    r = b + jnp.int32(0x7FFF) + (lax.shift_right_logical(b, 16) & jnp.int32(1))
    return lax.shift_right_logical(r, 16)


def _pack_i32(x):
    hi = _rnd_bf16_bits(x[:, :MP // 2])
    lo = _rnd_bf16_bits(x[:, MP // 2:])
    return lax.shift_left(hi, 16) | lo


def _norm_body(ls_ref, lt_ref, tn_ref, lsp_ref, ltp_ref):
    zpad = jnp.zeros((_BR0, MP - M), jnp.float32)
    lt = lt_ref[...]
    ssq = jnp.sum(lt * lt, axis=1, keepdims=True)
    nrm = jnp.maximum(jnp.sqrt(ssq), 1e-12)
    tn_ref[...] = (lt / nrm).astype(jnp.bfloat16)
    lsp_ref[...] = _pack_i32(jnp.concatenate([ls_ref[...], zpad], axis=1))
    ltp_ref[...] = _pack_i32(jnp.concatenate([lt, zpad], axis=1))


def _normalize(logits_s, logits_t):
    return pl.pallas_call(
        _norm_body,
        grid=(B // _BR0,),
        in_specs=[
            pl.BlockSpec((_BR0, M), lambda i: (i, 0)),
            pl.BlockSpec((_BR0, M), lambda i: (i, 0)),
        ],
        out_specs=[
            pl.BlockSpec((_BR0, M), lambda i: (i, 0)),
            pl.BlockSpec((_BR0, MP // 2), lambda i: (i, 0)),
            pl.BlockSpec((_BR0, MP // 2), lambda i: (i, 0)),
        ],
        out_shape=[
            jax.ShapeDtypeStruct((B, M), jnp.bfloat16),
            jax.ShapeDtypeStruct((B, MP // 2), jnp.int32),
            jax.ShapeDtypeStruct((B, MP // 2), jnp.int32),
        ],
    )(logits_s, logits_t)


_BR1 = 512


def _topk_body(row0, lhs_ref, tn_ref, o_ref):
    g = lax.dot_general(lhs_ref[...].astype(jnp.bfloat16), tn_ref[...],
                        (((1,), (1,)), ((), ())),
                        preferred_element_type=jnp.float32)
    r0 = pl.program_id(0) * _BR1 + row0
    col = lax.broadcasted_iota(jnp.int32, (_BR1, B), 1)
    row = lax.broadcasted_iota(jnp.int32, (_BR1, B), 0) + r0
    bits = lax.bitcast_convert_type(g, jnp.int32)
    key = bits ^ ((bits >> 31) & jnp.int32(0x7FFFFFFF))
    packed = (key & jnp.int32(~0xFFF)) | (jnp.int32(B - 1) - col)
    imin = jnp.int32(-0x80000000)
    packed = jnp.where(col == row, imin, packed)
    lane = lax.broadcasted_iota(jnp.int32, (_BR1, 128), 1)
    out = jnp.zeros((_BR1, 128), jnp.int32)
    for k in range(KNN):
        m = jnp.max(packed, axis=1, keepdims=True)
        am = jnp.int32(B - 1) - (m & jnp.int32(0xFFF))
        out = jnp.where(lane == k, am, out)
        packed = jnp.where(packed == m, imin, packed)
    o_ref[...] = out


def _top4(logits_t, t_norm, row0, nrows):
    blk0 = row0 // _BR1
    return pl.pallas_call(
        functools.partial(_topk_body, row0),
        grid=(nrows // _BR1,),
        in_specs=[
            pl.BlockSpec((_BR1, M), lambda i: (i + blk0, 0)),
            pl.BlockSpec((B, M), lambda i: (0, 0)),
        ],
        out_specs=pl.BlockSpec((_BR1, 128), lambda i: (i, 0)),
        out_shape=jax.ShapeDtypeStruct((nrows, 128), jnp.int32),
    )(logits_t, t_norm)


_CH = 8
_NB = 4
_LA = 2


def _make_gather(nrows):
    info = plsc.get_sparse_core_info()
    nw = info.num_cores * info.num_subcores
    bpw = nrows // nw
    nch = bpw // _CH
    mesh = plsc.VectorSubcoreMesh(core_axis_name="c", subcore_axis_name="s")

    @functools.partial(
        pl.kernel,
        mesh=mesh,
        out_type=(jax.ShapeDtypeStruct((nrows, MP // 2), jnp.int32),
                  jax.ShapeDtypeStruct((nrows, MP // 2), jnp.int32)),
        scratch_types=[
            pltpu.VMEM((bpw,), jnp.int32),
            pltpu.VMEM((_NB, _CH, MP // 2), jnp.int32),
            pltpu.VMEM((_NB, _CH, MP // 2), jnp.int32),
            pltpu.SemaphoreType.DMA,
            pltpu.SemaphoreType.DMA,
            pltpu.SemaphoreType.DMA,
            pltpu.SemaphoreType.DMA,
        ],
    )
    def gather(ts_hbm, tt_hbm, idx_hbm, os_hbm, ot_hbm,
               idx_v, bufs, buft, gsem_s, gsem_t, osem_s, osem_t):
        wid = lax.axis_index("s") * info.num_cores + lax.axis_index("c")
        base = wid * bpw
        pltpu.sync_copy(idx_hbm.at[pl.ds(base, bpw)], idx_v)

        def fire_gathers(c, slot):
            ic = idx_v.at[pl.ds(c * _CH, _CH)]
            pltpu.async_copy(ts_hbm.at[ic], bufs.at[slot], gsem_s)
            pltpu.async_copy(tt_hbm.at[ic], buft.at[slot], gsem_t)

        def drain_g(slot):
            pltpu.make_async_copy(ts_hbm.at[pl.ds(0, _CH)], bufs.at[slot],
                                  gsem_s).wait()
            pltpu.make_async_copy(tt_hbm.at[pl.ds(0, _CH)], buft.at[slot],
                                  gsem_t).wait()

        def drain_o(slot):
            pltpu.make_async_copy(bufs.at[slot], os_hbm.at[pl.ds(0, _CH)],
                                  osem_s).wait()
            pltpu.make_async_copy(buft.at[slot], ot_hbm.at[pl.ds(0, _CH)],
                                  osem_t).wait()

        for c in range(_LA):
            fire_gathers(c, c % _NB)

        def body(i, _):
            for j in range(_NB):
                c = i * _NB + j
                slot = j
                off = base + c * _CH
                drain_g(slot)
                pltpu.async_copy(bufs.at[slot], os_hbm.at[pl.ds(off, _CH)],
                                 osem_s)
                pltpu.async_copy(buft.at[slot], ot_hbm.at[pl.ds(off, _CH)],
                                 osem_t)
                cn = c + _LA

                @pl.when(cn >= _NB)
                def _():
                    drain_o((slot + _LA) % _NB)

                @pl.when(cn < nch)
                def _():
                    fire_gathers(cn, (slot + _LA) % _NB)

            return 0

        lax.fori_loop(0, nch // _NB, body, 0)
        for _ in range(_LA):
            drain_o(0)

    return gather


_BR3 = 256
_SCALE = 0.5 / (B * KNN)


def _js_body(ls_ref, lt_ref, gs_ref, gt_ref, o_ref, acc_ref):
    @pl.when(pl.program_id(0) == 0)
    def _():
        acc_ref[0, 0] = 0.0

    ls = ls_ref[...]
    lt = lt_ref[...]
    total = jnp.float32(0.0)
    def unpack(z):
        a = lax.bitcast_convert_type(z & jnp.int32(~0xFFFF), jnp.float32)
        b = lax.bitcast_convert_type(lax.shift_left(z, 16), jnp.float32)
        return jnp.concatenate([a, b], axis=1)[:, :M]

    for k in range(KNN):
        dq = ls - unpack(gs_ref[k])
        dp = lt - unpack(gt_ref[k])

        def softmax_logsoftmax(d):
            m = jnp.max(d, axis=1, keepdims=True)
            e = jnp.exp(d - m)
            s = jnp.sum(e, axis=1, keepdims=True)
            return e / s, (d - m) - jnp.log(s)

        q, logq = softmax_logsoftmax(dq)
        p, logp = softmax_logsoftmax(dp)
        logmean = jnp.log((q + p) * 0.5)
        contrib = q * (logq - logmean) + p * (logp - logmean)
        total = total + jnp.sum(contrib)
    acc_ref[0, 0] += total

    @pl.when(pl.program_id(0) == pl.num_programs(0) - 1)
    def _():
        o_ref[0, 0] = acc_ref[0, 0] * _SCALE


def _js_loss(ls, lt, gs, gt, row0, nrows):
    blk0 = row0 // _BR3
    out = pl.pallas_call(
        _js_body,
        grid=(nrows // _BR3,),
        in_specs=[
            pl.BlockSpec((_BR3, M), lambda i: (i + blk0, 0)),
            pl.BlockSpec((_BR3, M), lambda i: (i + blk0, 0)),
            pl.BlockSpec((KNN, _BR3, MP // 2), lambda i: (0, i, 0)),
            pl.BlockSpec((KNN, _BR3, MP // 2), lambda i: (0, i, 0)),
        ],
        out_specs=pl.BlockSpec(memory_space=pltpu.SMEM),
        out_shape=jax.ShapeDtypeStruct((1, 1), jnp.float32),
        scratch_shapes=[pltpu.SMEM((1, 1), jnp.float32)],
    )(ls, lt, gs, gt)
    return out[0, 0]


_NSPLIT = 2
_HB = B // _NSPLIT


def kernel(logits_s, logits_t):
    t_norm, ls_p, lt_p = _normalize(logits_s, logits_t)
    gather = _make_gather(_HB * KNN)

    partials = []
    for h in range(_NSPLIT):
        idx128 = _top4(logits_t, t_norm, h * _HB, _HB)
        flat_idx = idx128[:, :KNN].T.reshape(-1)
        gs_flat, gt_flat = gather(ls_p, lt_p, flat_idx)
        gs = gs_flat.reshape(KNN, _HB, MP // 2)
        gt = gt_flat.reshape(KNN, _HB, MP // 2)
        partials.append(_js_loss(logits_s, logits_t, gs, gt, h * _HB, _HB))
    return sum(partials)
